# Initial kernel scaffold; baseline (speedup 1.0000x reference)
import jax
import jax.numpy as jnp
from jax.experimental import pallas as pl


def kernel(inputs, embeddings):
    raise NotImplementedError("write your pallas kernel here")


# speedup vs baseline: 1.1020x; 1.1020x over previous
import functools

import jax
import jax.numpy as jnp
from jax import lax
from jax.experimental import pallas as pl
from jax.experimental.pallas import tpu as pltpu
from jax.experimental.pallas import tpu_sc as plsc

VOCAB = 1000000
EMBED = 32

_info = plsc.get_sparse_core_info()
_NC, _NS = _info.num_cores, _info.num_subcores
_NW = _NC * _NS

_B = 16384 * 50
_BPW = _B // _NW
_CHUNK = 1600
_STEPS = _BPW // _CHUNK


def _gather_kernel(idx_hbm, table_hbm, out_hbm, idx_v, rows_v, sem):
    wid = lax.axis_index("s") * _NC + lax.axis_index("c")
    base = wid * _BPW

    def step(j, carry):
        off = base + j * _CHUNK
        pltpu.sync_copy(idx_hbm.at[pl.ds(off, _CHUNK)], idx_v)
        pltpu.async_copy(table_hbm.at[idx_v], rows_v, sem).wait()
        pltpu.sync_copy(rows_v, out_hbm.at[pl.ds(off, _CHUNK)])
        return carry

    lax.fori_loop(0, _STEPS, step, 0)


@jax.jit
def _embed_lookup(idx_flat, table):
    mesh = plsc.VectorSubcoreMesh(core_axis_name="c", subcore_axis_name="s")
    kf = functools.partial(
        pl.kernel,
        mesh=mesh,
        out_type=jax.ShapeDtypeStruct((_B, EMBED), jnp.float32),
        scratch_types=[
            pltpu.VMEM((_CHUNK,), jnp.int32),
            pltpu.VMEM((_CHUNK, EMBED), jnp.float32),
            pltpu.SemaphoreType.DMA,
        ],
        compiler_params=pltpu.CompilerParams(use_tc_tiling_on_sc=False),
    )(_gather_kernel)
    return kf(idx_flat, table)


def kernel(inputs, embeddings):
    idx_flat = inputs.astype(jnp.int32).reshape(-1)
    out = _embed_lookup(idx_flat, embeddings)
    return out.reshape(inputs.shape[0], inputs.shape[1], EMBED)


# speedup vs baseline: 1.5049x; 1.3656x over previous
import functools

import jax
import jax.numpy as jnp
from jax import lax
from jax.experimental import pallas as pl
from jax.experimental.pallas import tpu as pltpu
from jax.experimental.pallas import tpu_sc as plsc

VOCAB = 1000000
EMBED = 32

_info = plsc.get_sparse_core_info()
_NC, _NS = _info.num_cores, _info.num_subcores
_NW = _NC * _NS

_NB = 16384
_NT = 50
_BT = _NB // 128
_BT_W = _BT // _NW
_CT = EMBED // 8


def _gather_kernel(idx_hbm, tab_hbm, out_hbm,
                   chunk_v, list0, list1, rows0, rows1, perm0, perm1,
                   gsem0, gsem1, wsem0, wsem1):
    wid = lax.axis_index("s") * _NC + lax.axis_index("c")
    iota = lax.broadcasted_iota(jnp.int32, (16,), 0)
    iota2 = iota + iota
    iota50 = iota * _NT

    def build(t, list_x):
        for mb in range(8):
            pos = mb * (16 * _NT) + iota50 + t
            v = plsc.load_gather(chunk_v, [pos])
            e = v + v
            plsc.store_scatter(list_x, [mb * 32 + iota2], e)
            plsc.store_scatter(list_x, [mb * 32 + iota2 + 1], e + 1)

    def permute(rows_x, perm_x):
        for lb in range(8):
            rowv0 = lb * 32 + iota2
            for hi in range(2):
                rowv = rowv0 + hi
                for lo in range(16):
                    c = hi * 16 + lo
                    colv = iota * 0 + lo
                    v = plsc.load_gather(rows_x, [rowv, colv])
                    perm_x[c // 8, c % 8, pl.ds(lb * 16, 16)] = v

    def bt_body(k, carry):
        bt = wid * _BT_W + k
        pltpu.sync_copy(idx_hbm.at[pl.ds(bt * (128 * _NT), 128 * _NT)],
                        chunk_v)
        build(0, list0)
        pltpu.async_copy(tab_hbm.at[list0], rows0, gsem0)

        def pair_body(i, carry2):
            t0 = 2 * i
            t1 = t0 + 1
            build(t1, list1)
            pltpu.async_copy(tab_hbm.at[list1], rows1, gsem1)
            pltpu.make_async_copy(tab_hbm.at[list0], rows0, gsem0).wait()

            @pl.when(i > 0)
            def _():
                pltpu.make_async_copy(perm0, out_hbm.at[t0, :, bt],
                                      wsem0).wait()
            permute(rows0, perm0)
            pltpu.async_copy(perm0, out_hbm.at[t0, :, bt], wsem0)

            @pl.when(i < _NT // 2 - 1)
            def _():
                build(t0 + 2, list0)
                pltpu.async_copy(tab_hbm.at[list0], rows0, gsem0)
            pltpu.make_async_copy(tab_hbm.at[list1], rows1, gsem1).wait()

            @pl.when(i > 0)
            def _():
                pltpu.make_async_copy(perm1, out_hbm.at[t1, :, bt],
                                      wsem1).wait()
            permute(rows1, perm1)
            pltpu.async_copy(perm1, out_hbm.at[t1, :, bt], wsem1)
            return carry2

        lax.fori_loop(0, _NT // 2, pair_body, 0)
        pltpu.make_async_copy(perm0, out_hbm.at[_NT - 2, :, bt], wsem0).wait()
        pltpu.make_async_copy(perm1, out_hbm.at[_NT - 1, :, bt], wsem1).wait()
        return carry

    lax.fori_loop(0, _BT_W, bt_body, 0)


@jax.jit
def _embed_lookup(idx_flat, tab16):
    mesh = plsc.VectorSubcoreMesh(core_axis_name="c", subcore_axis_name="s")
    kf = functools.partial(
        pl.kernel,
        mesh=mesh,
        out_type=jax.ShapeDtypeStruct((_NT, _CT, _BT, 8, 128), jnp.float32),
        scratch_types=[
            pltpu.VMEM((128 * _NT,), jnp.int32),
            pltpu.VMEM((256,), jnp.int32),
            pltpu.VMEM((256,), jnp.int32),
            pltpu.VMEM((256, 16), jnp.float32),
            pltpu.VMEM((256, 16), jnp.float32),
            pltpu.VMEM((_CT, 8, 128), jnp.float32),
            pltpu.VMEM((_CT, 8, 128), jnp.float32),
            pltpu.SemaphoreType.DMA,
            pltpu.SemaphoreType.DMA,
            pltpu.SemaphoreType.DMA,
            pltpu.SemaphoreType.DMA,
        ],
        compiler_params=pltpu.CompilerParams(
            use_tc_tiling_on_sc=False, needs_layout_passes=False),
    )(_gather_kernel)
    return kf(idx_flat, tab16)


def kernel(inputs, embeddings):
    idx_flat = inputs.astype(jnp.int32).reshape(-1)
    tab16 = embeddings.reshape(2 * VOCAB, 16)
    out5 = _embed_lookup(idx_flat, tab16)
    return out5.transpose(2, 4, 0, 1, 3).reshape(_NB, _NT, EMBED)


# speedup vs baseline: 2.0805x; 1.3824x over previous
import functools

import jax
import jax.numpy as jnp
from jax import lax
from jax.experimental import pallas as pl
from jax.experimental.pallas import tpu as pltpu
from jax.experimental.pallas import tpu_sc as plsc

VOCAB = 1000000
EMBED = 32

_info = plsc.get_sparse_core_info()
_NC, _NS = _info.num_cores, _info.num_subcores
_NW = _NC * _NS

_NB = 16384
_NT = 50
_BT = _NB // 128
_BT_W = _BT // _NW
_CT = EMBED // 8


def _gather_kernel(idx_hbm, tab_hbm, out_hbm,
                   chunk_v, list0, list1, rows0, rows1, perm0, perm1,
                   gsem0, gsem1, wsem0, wsem1):
    wid = lax.axis_index("s") * _NC + lax.axis_index("c")
    iota = lax.broadcasted_iota(jnp.int32, (16,), 0)
    iota2 = iota + iota
    iota50 = iota * _NT

    def build(t, list_x):
        for mb in range(8):
            pos = mb * (16 * _NT) + iota50 + t
            v = plsc.load_gather(chunk_v, [pos])
            e = v + v
            plsc.store_scatter(list_x, [mb * 32 + iota2], e)
            plsc.store_scatter(list_x, [mb * 32 + iota2 + 1], e + 1)

    colvs = [iota * 0 + lo for lo in range(16)]

    def permute(rows_x, perm_x):
        for lb in range(8):
            rowv0 = lb * 32 + iota2
            for hi in range(2):
                rowv = rowv0 + hi
                vs = [plsc.load_gather(rows_x, [rowv, colvs[lo]])
                      for lo in range(16)]
                for lo in range(16):
                    c = hi * 16 + lo
                    perm_x[c // 8,
                           pl.ds((c % 8) * 128 + lb * 16, 16)] = vs[lo]

    def bt_body(k, carry):
        bt = wid * _BT_W + k
        pltpu.sync_copy(idx_hbm.at[pl.ds(bt * (128 * _NT), 128 * _NT)],
                        chunk_v)
        build(0, list0)
        pltpu.async_copy(tab_hbm.at[list0], rows0, gsem0)

        def pair_body(i, carry2):
            t0 = 2 * i
            t1 = t0 + 1
            build(t1, list1)
            pltpu.async_copy(tab_hbm.at[list1], rows1, gsem1)
            pltpu.make_async_copy(tab_hbm.at[list0], rows0, gsem0).wait()

            @pl.when(i > 0)
            def _():
                pltpu.make_async_copy(perm0, out_hbm.at[t0, :, bt],
                                      wsem0).wait()
            permute(rows0, perm0)
            pltpu.async_copy(perm0, out_hbm.at[t0, :, bt], wsem0)

            @pl.when(i < _NT // 2 - 1)
            def _():
                build(t0 + 2, list0)
                pltpu.async_copy(tab_hbm.at[list0], rows0, gsem0)
            pltpu.make_async_copy(tab_hbm.at[list1], rows1, gsem1).wait()

            @pl.when(i > 0)
            def _():
                pltpu.make_async_copy(perm1, out_hbm.at[t1, :, bt],
                                      wsem1).wait()
            permute(rows1, perm1)
            pltpu.async_copy(perm1, out_hbm.at[t1, :, bt], wsem1)
            return carry2

        lax.fori_loop(0, _NT // 2, pair_body, 0)
        pltpu.make_async_copy(perm0, out_hbm.at[_NT - 2, :, bt], wsem0).wait()
        pltpu.make_async_copy(perm1, out_hbm.at[_NT - 1, :, bt], wsem1).wait()
        return carry

    lax.fori_loop(0, _BT_W, bt_body, 0)


@jax.jit
def _embed_lookup(idx_flat, tab16):
    mesh = plsc.VectorSubcoreMesh(core_axis_name="c", subcore_axis_name="s")
    kf = functools.partial(
        pl.kernel,
        mesh=mesh,
        out_type=jax.ShapeDtypeStruct((_NT, _CT, _BT, 1024), jnp.float32),
        scratch_types=[
            pltpu.VMEM((128 * _NT,), jnp.int32),
            pltpu.VMEM((256,), jnp.int32),
            pltpu.VMEM((256,), jnp.int32),
            pltpu.VMEM((256, 16), jnp.float32),
            pltpu.VMEM((256, 16), jnp.float32),
            pltpu.VMEM((_CT, 1024), jnp.float32),
            pltpu.VMEM((_CT, 1024), jnp.float32),
            pltpu.SemaphoreType.DMA,
            pltpu.SemaphoreType.DMA,
            pltpu.SemaphoreType.DMA,
            pltpu.SemaphoreType.DMA,
        ],
        compiler_params=pltpu.CompilerParams(
            use_tc_tiling_on_sc=False, needs_layout_passes=False),
    )(_gather_kernel)
    return kf(idx_flat, tab16)


def kernel(inputs, embeddings):
    idx_flat = inputs.astype(jnp.int32).reshape(-1)
    tab16 = embeddings.reshape(2 * VOCAB, 16)
    out5 = _embed_lookup(idx_flat, tab16)
    return (out5.reshape(_NT, _CT, _BT, 8, 128)
            .transpose(2, 4, 0, 1, 3).reshape(_NB, _NT, EMBED))


# speedup vs baseline: 2.1021x; 1.0104x over previous
import functools

import jax
import jax.numpy as jnp
from jax import lax
from jax.experimental import pallas as pl
from jax.experimental.pallas import tpu as pltpu
from jax.experimental.pallas import tpu_sc as plsc

VOCAB = 1000000
EMBED = 32

_info = plsc.get_sparse_core_info()
_NC, _NS = _info.num_cores, _info.num_subcores
_NW = _NC * _NS

_NB = 16384
_NT = 50
_BT = _NB // 128
_BT_W = _BT // _NW
_CT = EMBED // 8


def _gather_kernel(idx_hbm, tab_hbm, out_hbm,
                   chunk_v, list0, list1, rows0, rows1, perm0, perm1,
                   gsem0, gsem1, wsem0, wsem1):
    wid = lax.axis_index("s") * _NC + lax.axis_index("c")
    iota = lax.broadcasted_iota(jnp.int32, (16,), 0)
    iota2 = iota + iota
    iota50 = iota * _NT

    def build(t, list_x):
        for mb in range(8):
            pos = mb * (16 * _NT) + iota50 + t
            v = plsc.load_gather(chunk_v, [pos])
            list_x[pl.ds(mb * 16, 16)] = v

    colvs = [iota * 0 + lo for lo in range(16)]
    rowvs = [lb * 16 + iota for lb in range(8)]
    hi16 = iota * 0 + 16

    def permute(rows_x, perm_x):
        for lb in range(8):
            for hi in range(2):
                vs = [plsc.load_gather(
                          rows_x,
                          [rowvs[lb], colvs[lo] + hi16 if hi else colvs[lo]])
                      for lo in range(16)]
                for lo in range(16):
                    c = hi * 16 + lo
                    perm_x[c // 8,
                           pl.ds((c % 8) * 128 + lb * 16, 16)] = vs[lo]

    def bt_body(k, carry):
        bt = wid * _BT_W + k
        pltpu.sync_copy(idx_hbm.at[pl.ds(bt * (128 * _NT), 128 * _NT)],
                        chunk_v)
        build(0, list0)
        pltpu.async_copy(tab_hbm.at[list0], rows0, gsem0)

        def pair_body(i, carry2):
            t0 = 2 * i
            t1 = t0 + 1
            build(t1, list1)
            pltpu.async_copy(tab_hbm.at[list1], rows1, gsem1)
            pltpu.make_async_copy(tab_hbm.at[list0], rows0, gsem0).wait()

            @pl.when(i > 0)
            def _():
                pltpu.make_async_copy(perm0, out_hbm.at[t0, :, bt],
                                      wsem0).wait()
            permute(rows0, perm0)
            pltpu.async_copy(perm0, out_hbm.at[t0, :, bt], wsem0)

            @pl.when(i < _NT // 2 - 1)
            def _():
                build(t0 + 2, list0)
                pltpu.async_copy(tab_hbm.at[list0], rows0, gsem0)
            pltpu.make_async_copy(tab_hbm.at[list1], rows1, gsem1).wait()

            @pl.when(i > 0)
            def _():
                pltpu.make_async_copy(perm1, out_hbm.at[t1, :, bt],
                                      wsem1).wait()
            permute(rows1, perm1)
            pltpu.async_copy(perm1, out_hbm.at[t1, :, bt], wsem1)
            return carry2

        lax.fori_loop(0, _NT // 2, pair_body, 0)
        pltpu.make_async_copy(perm0, out_hbm.at[_NT - 2, :, bt], wsem0).wait()
        pltpu.make_async_copy(perm1, out_hbm.at[_NT - 1, :, bt], wsem1).wait()
        return carry

    lax.fori_loop(0, _BT_W, bt_body, 0)


@jax.jit
def _embed_lookup(idx_flat, tab16):
    mesh = plsc.VectorSubcoreMesh(core_axis_name="c", subcore_axis_name="s")
    kf = functools.partial(
        pl.kernel,
        mesh=mesh,
        out_type=jax.ShapeDtypeStruct((_NT, _CT, _BT, 1024), jnp.float32),
        scratch_types=[
            pltpu.VMEM((128 * _NT,), jnp.int32),
            pltpu.VMEM((128,), jnp.int32),
            pltpu.VMEM((128,), jnp.int32),
            pltpu.VMEM((128, EMBED), jnp.float32),
            pltpu.VMEM((128, EMBED), jnp.float32),
            pltpu.VMEM((_CT, 1024), jnp.float32),
            pltpu.VMEM((_CT, 1024), jnp.float32),
            pltpu.SemaphoreType.DMA,
            pltpu.SemaphoreType.DMA,
            pltpu.SemaphoreType.DMA,
            pltpu.SemaphoreType.DMA,
        ],
        compiler_params=pltpu.CompilerParams(
            use_tc_tiling_on_sc=False, needs_layout_passes=False),
    )(_gather_kernel)
    return kf(idx_flat, tab16)


def kernel(inputs, embeddings):
    idx_flat = inputs.astype(jnp.int32).reshape(-1)
    out5 = _embed_lookup(idx_flat, embeddings)
    return (out5.reshape(_NT, _CT, _BT, 8, 128)
            .transpose(2, 4, 0, 1, 3).reshape(_NB, _NT, EMBED))
